# baseline (speedup 1.0000x reference)
import functools

import jax
import jax.numpy as jnp
from jax import lax
from jax.experimental import pallas as pl
from jax.experimental.pallas import tpu as pltpu
from jax.experimental.pallas import tpu_sc as plsc

NUM_CORES = 2
NUM_SUBCORES = 16
NW = NUM_CORES * NUM_SUBCORES

CHUNK = 128
NBUF = 5


@functools.partial(jax.jit, static_argnums=(2, 3))
def _emb_lookup(idx, table, nchunk, dim):
    b_total = NW * nchunk * CHUNK
    b_per_w = nchunk * CHUNK
    rounds = nchunk // NBUF

    mesh = plsc.VectorSubcoreMesh(core_axis_name="c", subcore_axis_name="s")

    @functools.partial(
        pl.kernel,
        mesh=mesh,
        out_type=jax.ShapeDtypeStruct((b_total, dim), jnp.float32),
        scratch_types=[
            pltpu.VMEM((nchunk, CHUNK), jnp.int32),
            pltpu.VMEM((NBUF, CHUNK, dim), jnp.float32),
        ] + [pltpu.SemaphoreType.DMA] * NBUF,
        compiler_params=pltpu.CompilerParams(use_tc_tiling_on_sc=False),
    )
    def emb_kernel(idx_hbm, table_hbm, out_hbm, idx_v, rows_v, *gsems):
        wid = lax.axis_index("s") * NUM_CORES + lax.axis_index("c")
        base = wid * b_per_w
        pltpu.sync_copy(idx_hbm.at[wid], idx_v)

        for b in range(NBUF):
            pltpu.async_copy(table_hbm.at[idx_v.at[b]], rows_v.at[b], gsems[b])

        def body(i, carry):
            for b in range(NBUF):
                j = i * NBUF + b
                pltpu.make_async_copy(
                    table_hbm.at[idx_v.at[j]], rows_v.at[b], gsems[b]
                ).wait()
                pltpu.sync_copy(
                    rows_v.at[b], out_hbm.at[pl.ds(base + j * CHUNK, CHUNK)]
                )

                @pl.when(i < rounds - 1)
                def _():
                    pltpu.async_copy(
                        table_hbm.at[idx_v.at[j + NBUF]], rows_v.at[b], gsems[b]
                    )

            return carry

        lax.fori_loop(0, rounds, body, 0)

    return emb_kernel(idx, table)


def kernel(x, weight):
    b, t = x.shape
    dim = weight.shape[1]
    b_total = b * t
    assert b_total % (NW * CHUNK) == 0
    nchunk = b_total // (NW * CHUNK)
    assert nchunk % NBUF == 0
    idx = x.reshape(NW, nchunk, CHUNK).astype(jnp.int32)
    rows = _emb_lookup(idx, weight, nchunk, dim)
    return rows.reshape(b, t, dim)
